# baseline (speedup 1.0000x reference)
import functools

import jax
import jax.numpy as jnp
from jax import lax
from jax.experimental import pallas as pl
from jax.experimental.pallas import tpu as pltpu
from jax.experimental.pallas import tpu_sc as plsc

_LANES = 16
_NC = 2
_NS = 16
_NW = _NC * _NS
_CHUNK = 128


def _rsqrt_nr(v):
    i = plsc.bitcast(v, jnp.int32)
    i = jnp.int32(0x5F3759DF) - (i >> 1)
    r = plsc.bitcast(i, jnp.float32)
    half = v * 0.5
    for _ in range(3):
        r = r * (1.5 - half * r * r)
    return r


def _tree_sum(vals):
    vals = list(vals)
    while len(vals) > 1:
        vals = [vals[i] + vals[i + 1] for i in range(0, len(vals) - 1, 2)] + (
            [vals[-1]] if len(vals) % 2 else [])
    return vals[0]


def _build_sc_call(T, V, H):
    nvec = H // _LANES
    per_w = T // _NW
    nchunks = per_w // _CHUNK
    mesh = plsc.VectorSubcoreMesh(
        core_axis_name="c", subcore_axis_name="s",
        num_cores=_NC, num_subcores=_NS)

    @functools.partial(
        pl.kernel,
        out_type=jax.ShapeDtypeStruct((T, H), jnp.float32),
        mesh=mesh,
        compiler_params=pltpu.CompilerParams(needs_layout_passes=False),
        scratch_types=[
            pltpu.VMEM((V, H), jnp.float32),
            pltpu.VMEM((2, H), jnp.float32),
            pltpu.VMEM((2, _CHUNK), jnp.int32),
            pltpu.VMEM((2, _CHUNK, H), jnp.float32),
            pltpu.VMEM((2, _CHUNK, H), jnp.float32),
            pltpu.VMEM((_CHUNK, _LANES), jnp.float32),
            pltpu.VMEM((_CHUNK, _LANES), jnp.float32),
            pltpu.VMEM((_CHUNK, H // 2), jnp.int32),
            pltpu.SemaphoreType.DMA((2,)),
            pltpu.SemaphoreType.DMA((2,)),
            pltpu.SemaphoreType.DMA((2,)),
        ],
    )
    def sc_encode(x_hbm, ids_hbm, tab_hbm, g_hbm, b_hbm, out_hbm,
                  tab_v, gb_v, idall, xbuf, obuf, st2, qt2, ybuf,
                  isem, osem, idsem):
        wid = lax.axis_index("s") * _NC + lax.axis_index("c")
        base = wid * per_w
        pltpu.sync_copy(tab_hbm, tab_v)
        pltpu.sync_copy(g_hbm, gb_v.at[0])
        pltpu.sync_copy(b_hbm, gb_v.at[1])
        gvec = [gb_v[0, pl.ds(_LANES * j, _LANES)] for j in range(nvec)]
        bvec = [gb_v[1, pl.ds(_LANES * j, _LANES)] for j in range(nvec)]

        def in_cp(g, par):
            off = base + g * _CHUNK
            return pltpu.make_async_copy(
                x_hbm.at[pl.ds(off, _CHUNK), :], xbuf.at[par], isem.at[par])

        def ids_cp(g, par):
            off = base + g * _CHUNK
            return pltpu.make_async_copy(
                ids_hbm.at[pl.ds(off, _CHUNK)], idall.at[par], idsem.at[par])

        def out_cp(g, par):
            off = base + g * _CHUNK
            return pltpu.make_async_copy(
                obuf.at[par], out_hbm.at[pl.ds(off, _CHUNK), :], osem.at[par])

        rows_iota = lax.iota(jnp.int32, _LANES)

        def compute(g, par):
            @plsc.parallel_loop(0, _CHUNK // _LANES)
            def tok_group(tg):
                t0 = tg * _LANES
                ids_vec = idall[par, pl.ds(t0, _LANES)]
                for k in range(_LANES):
                    t = t0 + k
                    sid = ids_vec[k]
                    ys = []
                    for j in range(nvec):
                        xv = xbuf[par, t, pl.ds(_LANES * j, _LANES)]
                        ev = tab_v[sid, pl.ds(_LANES * j, _LANES)]
                        ys.append(xv + ev)
                    st2[t, pl.ds(0, _LANES)] = _tree_sum(ys)
                    qt2[t, pl.ds(0, _LANES)] = _tree_sum([y * y for y in ys])
                    for jj in range(nvec // 2):
                        packed = plsc.pack(ys[2 * jj], ys[2 * jj + 1],
                                           format=plsc.PackFormat.INTERLEAVED)
                        ybuf[t, pl.ds(_LANES * jj, _LANES)] = plsc.bitcast(
                            packed, jnp.int32)
                grows = rows_iota + t0
                scols = [plsc.load_gather(
                    st2, [grows, jnp.full((_LANES,), l, jnp.int32)])
                    for l in range(_LANES)]
                qcols = [plsc.load_gather(
                    qt2, [grows, jnp.full((_LANES,), l, jnp.int32)])
                    for l in range(_LANES)]
                mean_vec = _tree_sum(scols) * (1.0 / H)
                var_vec = _tree_sum(qcols) * (1.0 / H) - mean_vec * mean_vec
                rinv_vec = _rsqrt_nr(var_vec + 1e-5)
                for k in range(_LANES):
                    t = t0 + k
                    mean = lax.broadcast(mean_vec[k], (_LANES,))
                    rinv = lax.broadcast(rinv_vec[k], (_LANES,))
                    for jj in range(nvec // 2):
                        packed = plsc.bitcast(
                            ybuf[t, pl.ds(_LANES * jj, _LANES)], jnp.bfloat16)
                        ya, yb = plsc.unpack(
                            packed, format=plsc.PackFormat.INTERLEAVED)
                        for j, y in ((2 * jj, ya), (2 * jj + 1, yb)):
                            outv = (y - mean) * rinv * gvec[j] + bvec[j]
                            obuf[par, t, pl.ds(_LANES * j, _LANES)] = outv

        def step(g, par):
            in_cp(g, par).wait()
            ids_cp(g, par).wait()

            @pl.when(g >= 2)
            def _():
                out_cp(g - 2, par).wait()

            compute(g, par)
            out_cp(g, par).start()

            @pl.when(g + 2 < nchunks)
            def _():
                in_cp(g + 2, par).start()
                ids_cp(g + 2, par).start()

        in_cp(0, 0).start()
        ids_cp(0, 0).start()
        in_cp(1, 1).start()
        ids_cp(1, 1).start()

        def pair(i, c):
            step(2 * i, 0)
            step(2 * i + 1, 1)
            return c

        lax.fori_loop(0, nchunks // 2, pair, 0)
        out_cp(nchunks - 2, 0).wait()
        out_cp(nchunks - 1, 1).wait()

    return sc_encode


def kernel(batch_elem_emb, sent_pos_ids, emb_table, ln_gamma, ln_beta):
    B, N, H = batch_elem_emb.shape
    V = emb_table.shape[0]
    T = B * N
    x2d = jnp.transpose(batch_elem_emb, (1, 0, 2)).reshape(T, H)
    ids = jnp.transpose(sent_pos_ids, (1, 0)).reshape(T).astype(jnp.int32)
    sc_call = _build_sc_call(T, V, H)
    out = sc_call(x2d, ids, emb_table.astype(jnp.float32),
                  ln_gamma.astype(jnp.float32), ln_beta.astype(jnp.float32))
    return jnp.transpose(out.reshape(N, B, H), (1, 0, 2))


# speedup vs baseline: 1.0275x; 1.0275x over previous
import functools

import jax
import jax.numpy as jnp
from jax import lax
from jax.experimental import pallas as pl
from jax.experimental.pallas import tpu as pltpu
from jax.experimental.pallas import tpu_sc as plsc

_LANES = 16
_NC = 2
_NS = 16
_NW = _NC * _NS
_CHUNK = 128


def _rsqrt_nr(v):
    i = plsc.bitcast(v, jnp.int32)
    i = jnp.int32(0x5F3759DF) - (i >> 1)
    r = plsc.bitcast(i, jnp.float32)
    half = v * 0.5
    for _ in range(3):
        r = r * (1.5 - half * r * r)
    return r


def _tree_sum(vals):
    vals = list(vals)
    while len(vals) > 1:
        vals = [vals[i] + vals[i + 1] for i in range(0, len(vals) - 1, 2)] + (
            [vals[-1]] if len(vals) % 2 else [])
    return vals[0]


def _build_sc_call(T, V, H):
    nvec = H // _LANES
    per_w = T // _NW
    nchunks = per_w // _CHUNK
    mesh = plsc.VectorSubcoreMesh(
        core_axis_name="c", subcore_axis_name="s",
        num_cores=_NC, num_subcores=_NS)

    @functools.partial(
        pl.kernel,
        out_type=jax.ShapeDtypeStruct((T, H), jnp.float32),
        mesh=mesh,
        compiler_params=pltpu.CompilerParams(needs_layout_passes=False),
        scratch_types=[
            pltpu.VMEM((V, H), jnp.float32),
            pltpu.VMEM((2, H), jnp.float32),
            pltpu.VMEM((2, _CHUNK), jnp.int32),
            pltpu.VMEM((2, _CHUNK, H), jnp.float32),
            pltpu.VMEM((2, _CHUNK, H), jnp.float32),
            pltpu.VMEM((_CHUNK, _LANES), jnp.float32),
            pltpu.VMEM((_CHUNK, _LANES), jnp.float32),
            pltpu.SemaphoreType.DMA((2,)),
            pltpu.SemaphoreType.DMA((2,)),
            pltpu.SemaphoreType.DMA((2,)),
        ],
    )
    def sc_encode(x_hbm, ids_hbm, tab_hbm, g_hbm, b_hbm, out_hbm,
                  tab_v, gb_v, idall, xbuf, obuf, st2, qt2,
                  isem, osem, idsem):
        wid = lax.axis_index("s") * _NC + lax.axis_index("c")
        base = wid * per_w
        pltpu.sync_copy(tab_hbm, tab_v)
        pltpu.sync_copy(g_hbm, gb_v.at[0])
        pltpu.sync_copy(b_hbm, gb_v.at[1])
        gvec = [gb_v[0, pl.ds(_LANES * j, _LANES)] for j in range(nvec)]
        bvec = [gb_v[1, pl.ds(_LANES * j, _LANES)] for j in range(nvec)]

        def in_cp(g, par):
            off = base + g * _CHUNK
            return pltpu.make_async_copy(
                x_hbm.at[pl.ds(off, _CHUNK), :], xbuf.at[par], isem.at[par])

        def ids_cp(g, par):
            off = base + g * _CHUNK
            return pltpu.make_async_copy(
                ids_hbm.at[pl.ds(off, _CHUNK)], idall.at[par], idsem.at[par])

        def out_cp(g, par):
            off = base + g * _CHUNK
            return pltpu.make_async_copy(
                obuf.at[par], out_hbm.at[pl.ds(off, _CHUNK), :], osem.at[par])

        rows_iota = lax.iota(jnp.int32, _LANES)

        def compute(g, par):
            ngroups = _CHUNK // _LANES

            def pass1(tg):
                t0 = tg * _LANES
                ids_vec = idall[par, pl.ds(t0, _LANES)]
                for k in range(_LANES):
                    t = t0 + k
                    sid = ids_vec[k]
                    ys = []
                    for j in range(nvec):
                        xv = xbuf[par, t, pl.ds(_LANES * j, _LANES)]
                        ev = tab_v[sid, pl.ds(_LANES * j, _LANES)]
                        ys.append(xv + ev)
                    st2[t, pl.ds(0, _LANES)] = _tree_sum(ys)
                    qt2[t, pl.ds(0, _LANES)] = _tree_sum([y * y for y in ys])
                    for j in range(nvec):
                        obuf[par, t, pl.ds(_LANES * j, _LANES)] = ys[j]
                grows = rows_iota + t0
                scols = [plsc.load_gather(
                    st2, [grows, jnp.full((_LANES,), l, jnp.int32)])
                    for l in range(_LANES)]
                qcols = [plsc.load_gather(
                    qt2, [grows, jnp.full((_LANES,), l, jnp.int32)])
                    for l in range(_LANES)]
                mean_vec = _tree_sum(scols) * (1.0 / H)
                var_vec = _tree_sum(qcols) * (1.0 / H) - mean_vec * mean_vec
                rinv_vec = _rsqrt_nr(var_vec + 1e-5)
                return mean_vec, rinv_vec

            def pass2(tg, mean_vec, rinv_vec):
                t0 = tg * _LANES
                for k in range(_LANES):
                    t = t0 + k
                    mean = lax.broadcast(mean_vec[k], (_LANES,))
                    rinv = lax.broadcast(rinv_vec[k], (_LANES,))
                    for j in range(nvec):
                        y = obuf[par, t, pl.ds(_LANES * j, _LANES)]
                        outv = (y - mean) * rinv * gvec[j] + bvec[j]
                        obuf[par, t, pl.ds(_LANES * j, _LANES)] = outv

            def body(i, carry):
                m, r = carry
                nm, nr = pass1(i)
                pass2(i - 1, m, r)
                return nm, nr

            m0, r0 = pass1(0)
            m_l, r_l = lax.fori_loop(1, ngroups, body, (m0, r0))
            pass2(ngroups - 1, m_l, r_l)

        def step(g, par):
            in_cp(g, par).wait()
            ids_cp(g, par).wait()

            @pl.when(g >= 2)
            def _():
                out_cp(g - 2, par).wait()

            compute(g, par)
            out_cp(g, par).start()

            @pl.when(g + 2 < nchunks)
            def _():
                in_cp(g + 2, par).start()
                ids_cp(g + 2, par).start()

        in_cp(0, 0).start()
        ids_cp(0, 0).start()
        in_cp(1, 1).start()
        ids_cp(1, 1).start()

        def pair(i, c):
            step(2 * i, 0)
            step(2 * i + 1, 1)
            return c

        lax.fori_loop(0, nchunks // 2, pair, 0)
        out_cp(nchunks - 2, 0).wait()
        out_cp(nchunks - 1, 1).wait()

    return sc_encode


def kernel(batch_elem_emb, sent_pos_ids, emb_table, ln_gamma, ln_beta):
    B, N, H = batch_elem_emb.shape
    V = emb_table.shape[0]
    T = B * N
    x2d = jnp.transpose(batch_elem_emb, (1, 0, 2)).reshape(T, H)
    ids = jnp.transpose(sent_pos_ids, (1, 0)).reshape(T).astype(jnp.int32)
    sc_call = _build_sc_call(T, V, H)
    out = sc_call(x2d, ids, emb_table.astype(jnp.float32),
                  ln_gamma.astype(jnp.float32), ln_beta.astype(jnp.float32))
    return jnp.transpose(out.reshape(N, B, H), (1, 0, 2))


# speedup vs baseline: 1.1209x; 1.0909x over previous
import functools

import jax
import jax.numpy as jnp
from jax import lax
from jax.experimental import pallas as pl
from jax.experimental.pallas import tpu as pltpu
from jax.experimental.pallas import tpu_sc as plsc

_LANES = 16
_NC = 2
_NS = 16
_NW = _NC * _NS
_CHUNK = 128


def _rsqrt_nr(v):
    i = plsc.bitcast(v, jnp.int32)
    i = jnp.int32(0x5F3759DF) - (i >> 1)
    r = plsc.bitcast(i, jnp.float32)
    half = v * 0.5
    for _ in range(3):
        r = r * (1.5 - half * r * r)
    return r


def _tree_sum(vals):
    vals = list(vals)
    while len(vals) > 1:
        vals = [vals[i] + vals[i + 1] for i in range(0, len(vals) - 1, 2)] + (
            [vals[-1]] if len(vals) % 2 else [])
    return vals[0]


def _build_sc_call(T, V, H):
    nvec = H // _LANES
    per_w = T // _NW
    nchunks = per_w // _CHUNK
    mesh = plsc.VectorSubcoreMesh(
        core_axis_name="c", subcore_axis_name="s",
        num_cores=_NC, num_subcores=_NS)

    @functools.partial(
        pl.kernel,
        out_type=jax.ShapeDtypeStruct((T, H), jnp.float32),
        mesh=mesh,
        compiler_params=pltpu.CompilerParams(needs_layout_passes=False),
        scratch_types=[
            pltpu.VMEM((V, H), jnp.float32),
            pltpu.VMEM((2, H), jnp.float32),
            pltpu.VMEM((2, _CHUNK), jnp.int32),
            pltpu.VMEM((2, _CHUNK, H), jnp.float32),
            pltpu.VMEM((2, _CHUNK, H), jnp.float32),
            pltpu.VMEM((_CHUNK, _LANES), jnp.float32),
            pltpu.VMEM((_CHUNK, _LANES), jnp.float32),
            pltpu.SemaphoreType.DMA((2,)),
            pltpu.SemaphoreType.DMA((2,)),
            pltpu.SemaphoreType.DMA((2,)),
        ],
    )
    def sc_encode(x_hbm, ids_hbm, tab_hbm, g_hbm, b_hbm, out_hbm,
                  tab_v, gb_v, idall, xbuf, obuf, st2, qt2,
                  isem, osem, idsem):
        wid = lax.axis_index("s") * _NC + lax.axis_index("c")
        base = wid * per_w
        pltpu.sync_copy(tab_hbm, tab_v)
        pltpu.sync_copy(g_hbm, gb_v.at[0])
        pltpu.sync_copy(b_hbm, gb_v.at[1])
        gvec = [gb_v[0, pl.ds(_LANES * j, _LANES)] for j in range(nvec)]
        bvec = [gb_v[1, pl.ds(_LANES * j, _LANES)] for j in range(nvec)]

        def in_cp(g, par):
            off = base + g * _CHUNK
            return pltpu.make_async_copy(
                x_hbm.at[pl.ds(off, _CHUNK), :], xbuf.at[par], isem.at[par])

        def ids_cp(g, par):
            off = base + g * _CHUNK
            return pltpu.make_async_copy(
                ids_hbm.at[pl.ds(off, _CHUNK)], idall.at[par], idsem.at[par])

        def out_cp(g, par):
            off = base + g * _CHUNK
            return pltpu.make_async_copy(
                obuf.at[par], out_hbm.at[pl.ds(off, _CHUNK), :], osem.at[par])

        rows_iota = lax.iota(jnp.int32, _LANES)

        def compute(g, par):
            @plsc.parallel_loop(0, _CHUNK // _LANES)
            def tok_group(tg):
                t0 = tg * _LANES
                ids_vec = idall[par, pl.ds(t0, _LANES)]
                for k in range(_LANES):
                    t = t0 + k
                    sid = ids_vec[k]
                    ys = []
                    for j in range(nvec):
                        xv = xbuf[par, t, pl.ds(_LANES * j, _LANES)]
                        ev = tab_v[sid, pl.ds(_LANES * j, _LANES)]
                        ys.append(xv + ev)
                    st2[t, pl.ds(0, _LANES)] = _tree_sum(ys)
                    qt2[t, pl.ds(0, _LANES)] = _tree_sum([y * y for y in ys])
                    for j in range(nvec):
                        obuf[par, t, pl.ds(_LANES * j, _LANES)] = ys[j]
                grows = rows_iota + t0
                scols = [plsc.load_gather(
                    st2, [grows, jnp.full((_LANES,), l, jnp.int32)])
                    for l in range(_LANES)]
                qcols = [plsc.load_gather(
                    qt2, [grows, jnp.full((_LANES,), l, jnp.int32)])
                    for l in range(_LANES)]
                mean_vec = _tree_sum(scols) * (1.0 / H)
                var_vec = _tree_sum(qcols) * (1.0 / H) - mean_vec * mean_vec
                rinv_vec = _rsqrt_nr(var_vec + 1e-5)
                for k in range(_LANES):
                    t = t0 + k
                    mean = lax.broadcast(mean_vec[k], (_LANES,))
                    rinv = lax.broadcast(rinv_vec[k], (_LANES,))
                    for j in range(nvec):
                        y = obuf[par, t, pl.ds(_LANES * j, _LANES)]
                        outv = (y - mean) * rinv * gvec[j] + bvec[j]
                        obuf[par, t, pl.ds(_LANES * j, _LANES)] = outv

        def step(g, par):
            in_cp(g, par).wait()
            ids_cp(g, par).wait()

            @pl.when(g >= 2)
            def _():
                out_cp(g - 2, par).wait()

            compute(g, par)
            out_cp(g, par).start()

            @pl.when(g + 2 < nchunks)
            def _():
                in_cp(g + 2, par).start()
                ids_cp(g + 2, par).start()

        in_cp(0, 0).start()
        ids_cp(0, 0).start()
        in_cp(1, 1).start()
        ids_cp(1, 1).start()

        def pair(i, c):
            step(2 * i, 0)
            step(2 * i + 1, 1)
            return c

        lax.fori_loop(0, nchunks // 2, pair, 0)
        out_cp(nchunks - 2, 0).wait()
        out_cp(nchunks - 1, 1).wait()

    return sc_encode


def kernel(batch_elem_emb, sent_pos_ids, emb_table, ln_gamma, ln_beta):
    B, N, H = batch_elem_emb.shape
    V = emb_table.shape[0]
    T = B * N
    x2d = jnp.transpose(batch_elem_emb, (1, 0, 2)).reshape(T, H)
    ids = jnp.transpose(sent_pos_ids, (1, 0)).reshape(T).astype(jnp.int32)
    sc_call = _build_sc_call(T, V, H)
    out = sc_call(x2d, ids, emb_table.astype(jnp.float32),
                  ln_gamma.astype(jnp.float32), ln_beta.astype(jnp.float32))
    return jnp.transpose(out.reshape(N, B, H), (1, 0, 2))
